# Initial kernel scaffold; baseline (speedup 1.0000x reference)
import jax
import jax.numpy as jnp
from jax.experimental import pallas as pl


def kernel(x, x_0, edge_index, W, bias):
    raise NotImplementedError("write your pallas kernel here")


# speedup vs baseline: 7.0985x; 7.0985x over previous
import functools

import jax
import jax.numpy as jnp
from jax import lax
from jax.experimental import pallas as pl
from jax.experimental.pallas import tpu as pltpu
from jax.experimental.pallas import tpu_sc as plsc

N = 10000
C = 128
K = 32
DEG = K + 1

NW = 32
NPT = 320
N_PAD = NW * NPT
CH = 4
ROWS = CH * K
NCH = NPT // CH
MM_NB = 2048


def _mm_body(x_ref, w_ref, o_ref):
    h = lax.dot_general(x_ref[...], w_ref[...], (((0,), (1,)), ((), ())),
                        preferred_element_type=jnp.float32)
    o_ref[...] = jnp.maximum(h, 0.0)


def _matmul_relu(x2d, W):
    grid = (N_PAD // MM_NB,)
    return pl.pallas_call(
        _mm_body,
        grid=grid,
        in_specs=[
            pl.BlockSpec((C, MM_NB), lambda i: (0, i)),
            pl.BlockSpec((C, C), lambda i: (0, 0)),
        ],
        out_specs=pl.BlockSpec((MM_NB, C), lambda i: (i, 0)),
        out_shape=jax.ShapeDtypeStruct((N_PAD, C), jnp.float32),
    )(x2d, W)


def _tr_body(a_ref, b_ref, o_ref):
    o_ref[...] = a_ref[...].T + b_ref[...]


def _transpose_bias(a, bias2d):
    grid = (N_PAD // MM_NB,)
    return pl.pallas_call(
        _tr_body,
        grid=grid,
        in_specs=[
            pl.BlockSpec((MM_NB, C), lambda i: (i, 0)),
            pl.BlockSpec((C, 1), lambda i: (0, 0)),
        ],
        out_specs=pl.BlockSpec((C, MM_NB), lambda i: (0, i)),
        out_shape=jax.ShapeDtypeStruct((C, N_PAD), jnp.float32),
    )(a, bias2d)


def _sc_body(h_hbm, idx_hbm, out_hbm, idx_v, self_v, out_v, gbuf, sem0, sem1):
    wid = lax.axis_index("s") * 2 + lax.axis_index("c")
    base = wid * NPT
    pltpu.sync_copy(idx_hbm.at[pl.ds(wid * NCH, NCH)], idx_v)
    pltpu.sync_copy(h_hbm.at[pl.ds(base, NPT)], self_v)

    def gather(chunk, buf, sem):
        pltpu.make_async_copy(h_hbm.at[idx_v.at[chunk]], gbuf.at[buf], sem
                              ).start()

    def wait(buf, sem):
        pltpu.make_async_copy(h_hbm.at[idx_v.at[0]], gbuf.at[buf], sem).wait()

    def compute(chunk, buf):
        for cn in range(CH):
            node = chunk * CH + cn
            for g in range(C // 16):
                sl = pl.ds(g * 16, 16)
                acc = self_v[node, sl]
                for k in range(K):
                    acc = acc + gbuf[buf, cn * K + k, sl]
                out_v[node, sl] = acc * (1.0 / DEG)

    gather(0, 0, sem0)

    def body(i, carry):
        c0 = i * 2
        gather(c0 + 1, 1, sem1)
        wait(0, sem0)
        compute(c0, 0)

        @pl.when(c0 + 2 < NCH)
        def _():
            gather(c0 + 2, 0, sem0)

        wait(1, sem1)
        compute(c0 + 1, 1)
        return carry

    lax.fori_loop(0, NCH // 2, body, 0)
    pltpu.sync_copy(out_v, out_hbm.at[pl.ds(base, NPT)])


_sc_aggregate = functools.partial(
    pl.kernel,
    out_type=jax.ShapeDtypeStruct((N_PAD, C), jnp.float32),
    mesh=plsc.VectorSubcoreMesh(core_axis_name="c", subcore_axis_name="s"),
    scratch_types=[
        pltpu.VMEM((NCH, ROWS), jnp.int32),
        pltpu.VMEM((NPT, C), jnp.float32),
        pltpu.VMEM((NPT, C), jnp.float32),
        pltpu.VMEM((2, ROWS, C), jnp.float32),
        pltpu.SemaphoreType.DMA,
        pltpu.SemaphoreType.DMA,
    ],
)(_sc_body)


def kernel(x, x_0, edge_index, W, bias):
    del x_0
    x2d = x[0, :, :, 0]
    x2d = jnp.pad(x2d, ((0, 0), (0, N_PAD - N)))
    h = _matmul_relu(x2d, W)

    idx = edge_index[0, 0]
    idx = jnp.pad(idx, ((0, N_PAD - N), (0, 0)))
    idx = idx.reshape(NW * NCH, ROWS)

    aggr = _sc_aggregate(h, idx)

    bias2d = bias.reshape(C, 1)
    out = _transpose_bias(aggr, bias2d)
    return out[:, :N].reshape(1, C, N, 1)


# speedup vs baseline: 7.1040x; 1.0008x over previous
import functools

import jax
import jax.numpy as jnp
from jax import lax
from jax.experimental import pallas as pl
from jax.experimental.pallas import tpu as pltpu
from jax.experimental.pallas import tpu_sc as plsc

N = 10000
C = 128
K = 32
DEG = K + 1

NW = 32
NPT = 320
N_PAD = NW * NPT
CH = 4
ROWS = CH * K
NCH = NPT // CH
MM_NB = 2048


def _mm_body(x_ref, w_ref, o_ref):
    h = lax.dot_general(x_ref[...], w_ref[...], (((0,), (1,)), ((), ())),
                        preferred_element_type=jnp.float32)
    o_ref[...] = jnp.maximum(h, 0.0)


def _matmul_relu(x2d, W):
    grid = (N_PAD // MM_NB,)
    return pl.pallas_call(
        _mm_body,
        grid=grid,
        in_specs=[
            pl.BlockSpec((C, MM_NB), lambda i: (0, i)),
            pl.BlockSpec((C, C), lambda i: (0, 0)),
        ],
        out_specs=pl.BlockSpec((MM_NB, C), lambda i: (i, 0)),
        out_shape=jax.ShapeDtypeStruct((N_PAD, C), jnp.float32),
    )(x2d, W)


def _tr_body(a_ref, b_ref, o_ref):
    o_ref[...] = a_ref[...].T + b_ref[...]


def _transpose_bias(a, bias2d):
    grid = (N_PAD // MM_NB,)
    return pl.pallas_call(
        _tr_body,
        grid=grid,
        in_specs=[
            pl.BlockSpec((MM_NB, C), lambda i: (i, 0)),
            pl.BlockSpec((C, 1), lambda i: (0, 0)),
        ],
        out_specs=pl.BlockSpec((C, MM_NB), lambda i: (0, i)),
        out_shape=jax.ShapeDtypeStruct((C, N_PAD), jnp.float32),
    )(a, bias2d)


def _sc_body(h_hbm, idx_hbm, out_hbm, idx_v, self_v, out_v, gbuf, sem0, sem1):
    wid = lax.axis_index("s") * 2 + lax.axis_index("c")
    base = wid * NPT
    pltpu.sync_copy(idx_hbm.at[pl.ds(wid * NCH, NCH)], idx_v)
    pltpu.sync_copy(h_hbm.at[pl.ds(base, NPT)], self_v)

    def gather(chunk, buf, sem):
        pltpu.make_async_copy(h_hbm.at[idx_v.at[chunk]], gbuf.at[buf], sem
                              ).start()

    def wait(buf, sem):
        pltpu.make_async_copy(h_hbm.at[idx_v.at[0]], gbuf.at[buf], sem).wait()

    def compute(chunk, buf):
        for cn in range(CH):
            node = chunk * CH + cn
            for gh in range(2):
                gs = [gh * 4 + g for g in range(4)]
                accs = [self_v[node, pl.ds(g * 16, 16)] for g in gs]
                for k in range(K):
                    r = cn * K + k
                    for j, g in enumerate(gs):
                        accs[j] = accs[j] + gbuf[buf, r, pl.ds(g * 16, 16)]
                for j, g in enumerate(gs):
                    out_v[node, pl.ds(g * 16, 16)] = accs[j] * (1.0 / DEG)

    gather(0, 0, sem0)

    def body(i, carry):
        c0 = i * 2
        gather(c0 + 1, 1, sem1)
        wait(0, sem0)
        compute(c0, 0)

        @pl.when(c0 + 2 < NCH)
        def _():
            gather(c0 + 2, 0, sem0)

        wait(1, sem1)
        compute(c0 + 1, 1)
        return carry

    lax.fori_loop(0, NCH // 2, body, 0)
    pltpu.sync_copy(out_v, out_hbm.at[pl.ds(base, NPT)])


_sc_aggregate = functools.partial(
    pl.kernel,
    out_type=jax.ShapeDtypeStruct((N_PAD, C), jnp.float32),
    mesh=plsc.VectorSubcoreMesh(core_axis_name="c", subcore_axis_name="s"),
    scratch_types=[
        pltpu.VMEM((NCH, ROWS), jnp.int32),
        pltpu.VMEM((NPT, C), jnp.float32),
        pltpu.VMEM((NPT, C), jnp.float32),
        pltpu.VMEM((2, ROWS, C), jnp.float32),
        pltpu.SemaphoreType.DMA,
        pltpu.SemaphoreType.DMA,
    ],
)(_sc_body)


def kernel(x, x_0, edge_index, W, bias):
    del x_0
    x2d = x[0, :, :, 0]
    x2d = jnp.pad(x2d, ((0, 0), (0, N_PAD - N)))
    h = _matmul_relu(x2d, W)

    idx = edge_index[0, 0]
    idx = jnp.pad(idx, ((0, N_PAD - N), (0, 0)))
    idx = idx.reshape(NW * NCH, ROWS)

    aggr = _sc_aggregate(h, idx)

    bias2d = bias.reshape(C, 1)
    out = _transpose_bias(aggr, bias2d)
    return out[:, :N].reshape(1, C, N, 1)
